# Initial kernel scaffold; baseline (speedup 1.0000x reference)
import jax
import jax.numpy as jnp
from jax.experimental import pallas as pl


def kernel(atomic_numbers, displacements, idx_i, idx_j, atom_mask, batch_segments, batch_mask, batch_size):
    raise NotImplementedError("write your pallas kernel here")


# speedup vs baseline: 11.3825x; 11.3825x over previous
import functools

import jax
import jax.numpy as jnp
import numpy as np
from jax import lax
from jax.experimental import pallas as pl
from jax.experimental.pallas import tpu as pltpu
from jax.experimental.pallas import tpu_sc as plsc

NC = 2
NS = 16
L = 16
CHUNK = 2048

_PHI_C = np.abs(np.array([0.18175, 0.50986, 0.28022, 0.02817], np.float32))
_PHI_E = np.abs(np.array([3.1998, 0.94229, 0.4029, 0.20162], np.float32))
_SOFT = np.exp(_PHI_C - np.max(_PHI_C))
_COEF = (_SOFT / np.sum(_SOFT)).astype(np.float32)
_AEXP = (_PHI_E - _PHI_E[3]).astype(np.float32)
_INV_A = np.float32(1.0) / np.float32(0.8854)


def _zbl_body(zn_hbm, disp_hbm, ii_hbm, ij_hbm, zat_hbm, zero_hbm, out_hbm,
              ztab, zatab, iib, ijb, db, repb, acc, n_nodes, n_edges):
    cid = lax.axis_index("c")
    sid = lax.axis_index("s")
    wid = cid * NS + sid
    ept = n_edges // (NC * NS)
    nfull = ept // CHUNK
    tail = ept - nfull * CHUNK
    base0 = wid * ept

    pltpu.sync_copy(zn_hbm, ztab)
    pltpu.sync_copy(zat_hbm, zatab)

    @pl.when(sid == 0)
    def _():
        pltpu.sync_copy(zero_hbm, acc)

    plsc.subcore_barrier()

    iota = lax.iota(jnp.int32, L)
    iota3 = iota * 3

    def compute_vec(j):
        b16 = j * L
        ii = iib[pl.ds(b16, L)]
        ij = ijb[pl.ds(b16, L)]
        i0 = j * (3 * L) + iota3
        dx = plsc.load_gather(db, [i0])
        dy = plsc.load_gather(db, [i0 + 1])
        dz = plsc.load_gather(db, [i0 + 2])
        zi = plsc.load_gather(ztab, [ii])
        zj = plsc.load_gather(ztab, [ij])
        zai = plsc.load_gather(zatab, [zi])
        zaj = plsc.load_gather(zatab, [zj])
        d2 = jnp.maximum(dx * dx + dy * dy + dz * dz, jnp.float32(1e-20))
        bits = lax.bitcast_convert_type(d2, jnp.int32)
        y = lax.bitcast_convert_type(
            jnp.int32(0x5F3759DF) - lax.shift_right_arithmetic(bits, 1),
            jnp.float32)
        half = jnp.float32(0.5) * d2
        for _ in range(3):
            y = y * (jnp.float32(1.5) - half * y * y)
        dist = d2 * y
        arg = dist * (zai + zaj) * _INV_A
        phi = (_COEF[0] * jnp.exp(-_AEXP[0] * arg)
               + _COEF[1] * jnp.exp(-_AEXP[1] * arg)
               + _COEF[2] * jnp.exp(-_AEXP[2] * arg)
               + _COEF[3])
        x = jnp.float32(5.0) - dist
        sw = ((jnp.float32(6.0) * x - jnp.float32(15.0)) * x
              + jnp.float32(10.0)) * x * x * x
        sw = jnp.where(dist < jnp.float32(4.0), jnp.float32(1.0),
                       jnp.where(dist >= jnp.float32(5.0), jnp.float32(0.0),
                                 sw))
        sw = jnp.maximum(sw, jnp.float32(1e-30))
        zif = zi.astype(jnp.float32)
        zjf = zj.astype(jnp.float32)
        rep = (jnp.float32(0.5) * zif * zjf) * phi * sw * y
        repb[pl.ds(b16, L)] = rep

    def do_chunk(base, csize):
        pltpu.sync_copy(ii_hbm.at[pl.ds(base, csize)],
                        iib.at[pl.ds(0, csize)])
        pltpu.sync_copy(ij_hbm.at[pl.ds(base, csize)],
                        ijb.at[pl.ds(0, csize)])
        pltpu.sync_copy(disp_hbm.at[pl.ds(base * 3, csize * 3)],
                        db.at[pl.ds(0, csize * 3)])

        def vec_body(j, carry):
            compute_vec(j)
            return carry

        lax.fori_loop(0, csize // L, vec_body, 0)
        if csize < CHUNK:
            zf = jnp.zeros((L,), jnp.float32)
            zidx = jnp.zeros((L,), jnp.int32)
            for t in range((CHUNK - csize) // L):
                off = csize + t * L
                repb[pl.ds(off, L)] = zf
                iib[pl.ds(off, L)] = zidx
        pltpu.sync_copy(repb, acc.at[iib], add=True)

    def chunk_body(c, carry):
        do_chunk(base0 + c * CHUNK, CHUNK)
        return carry

    lax.fori_loop(0, nfull, chunk_body, 0)
    if tail:
        do_chunk(base0 + nfull * CHUNK, tail)

    plsc.subcore_barrier()

    @pl.when(sid == 0)
    def _():
        pltpu.sync_copy(acc, out_hbm.at[cid])


def kernel(atomic_numbers, displacements, idx_i, idx_j, atom_mask,
           batch_segments, batch_mask, batch_size):
    n_nodes = atomic_numbers.shape[0]
    n_edges = idx_i.shape[0]
    zn = atomic_numbers.astype(jnp.int32)
    disp_flat = displacements.reshape(-1)
    zat = jnp.power(jnp.arange(128, dtype=jnp.float32), jnp.float32(0.23))
    zeros_nodes = jnp.zeros((n_nodes,), jnp.float32)

    body = functools.partial(_zbl_body, n_nodes=n_nodes, n_edges=n_edges)
    run = pl.kernel(
        body,
        mesh=plsc.VectorSubcoreMesh(core_axis_name="c", subcore_axis_name="s"),
        out_type=jax.ShapeDtypeStruct((NC, n_nodes), jnp.float32),
        compiler_params=pltpu.CompilerParams(needs_layout_passes=False),
        scratch_types=[
            pltpu.VMEM((n_nodes,), jnp.int32),
            pltpu.VMEM((128,), jnp.float32),
            pltpu.VMEM((CHUNK,), jnp.int32),
            pltpu.VMEM((CHUNK,), jnp.int32),
            pltpu.VMEM((3 * CHUNK,), jnp.float32),
            pltpu.VMEM((CHUNK,), jnp.float32),
            pltpu.VMEM_SHARED((n_nodes,), jnp.float32),
        ],
    )
    partial = run(zn, disp_flat, idx_i.astype(jnp.int32),
                  idx_j.astype(jnp.int32), zat, zeros_nodes)
    erep = (partial[0] + partial[1]) * atom_mask
    return erep[..., None, None, None]


# speedup vs baseline: 49.5023x; 4.3490x over previous
import functools

import jax
import jax.numpy as jnp
import numpy as np
from jax import lax
from jax.experimental import pallas as pl
from jax.experimental.pallas import tpu as pltpu
from jax.experimental.pallas import tpu_sc as plsc

NC = 2
NS = 16
L = 16
CHUNK = 2048

_PHI_C = np.abs(np.array([0.18175, 0.50986, 0.28022, 0.02817], np.float32))
_PHI_E = np.abs(np.array([3.1998, 0.94229, 0.4029, 0.20162], np.float32))
_SOFT = np.exp(_PHI_C - np.max(_PHI_C))
_COEF = (_SOFT / np.sum(_SOFT)).astype(np.float32)
_AEXP = (_PHI_E - _PHI_E[3]).astype(np.float32)
_INV_A = np.float32(1.0) / np.float32(0.8854)


def _d2_tc_kernel(x_ref, o_ref):
    x = x_ref[...]
    o_ref[...] = jnp.sum(x * x, axis=1)


def _edge_d2(displacements):
    n_edges = displacements.shape[0]
    blk = 8192
    return pl.pallas_call(
        _d2_tc_kernel,
        grid=(n_edges // blk,),
        in_specs=[pl.BlockSpec((blk, 3), lambda i: (i, 0))],
        out_specs=pl.BlockSpec((blk,), lambda i: (i,)),
        out_shape=jax.ShapeDtypeStruct((n_edges,), jnp.float32),
    )(displacements)


def _zbl_body(zn_hbm, d2_hbm, ii_hbm, ij_hbm, zat_hbm, zero_hbm, out_hbm,
              ztab, zatab, iib, ijb, db, repb, acc, n_nodes, n_edges):
    cid = lax.axis_index("c")
    sid = lax.axis_index("s")
    wid = cid * NS + sid
    ept = n_edges // (NC * NS)
    nfull = ept // CHUNK
    tail = ept - nfull * CHUNK
    base0 = wid * ept

    pltpu.sync_copy(zn_hbm, ztab)
    pltpu.sync_copy(zat_hbm, zatab)

    @pl.when(sid == 0)
    def _():
        pltpu.sync_copy(zero_hbm, acc)

    plsc.subcore_barrier()

    def compute_vec(j):
        b16 = j * L
        ii = iib[pl.ds(b16, L)]
        ij = ijb[pl.ds(b16, L)]
        zi = plsc.load_gather(ztab, [ii])
        zj = plsc.load_gather(ztab, [ij])
        zai = plsc.load_gather(zatab, [zi])
        zaj = plsc.load_gather(zatab, [zj])
        d2 = jnp.maximum(db[pl.ds(b16, L)], jnp.float32(1e-20))
        bits = lax.bitcast_convert_type(d2, jnp.int32)
        y = lax.bitcast_convert_type(
            jnp.int32(0x5F3759DF) - lax.shift_right_arithmetic(bits, 1),
            jnp.float32)
        half = jnp.float32(0.5) * d2
        for _ in range(3):
            y = y * (jnp.float32(1.5) - half * y * y)
        dist = d2 * y
        arg = dist * (zai + zaj) * _INV_A
        phi = (_COEF[0] * jnp.exp(-_AEXP[0] * arg)
               + _COEF[1] * jnp.exp(-_AEXP[1] * arg)
               + _COEF[2] * jnp.exp(-_AEXP[2] * arg)
               + _COEF[3])
        x = jnp.float32(5.0) - dist
        sw = ((jnp.float32(6.0) * x - jnp.float32(15.0)) * x
              + jnp.float32(10.0)) * x * x * x
        sw = jnp.where(dist < jnp.float32(4.0), jnp.float32(1.0),
                       jnp.where(dist >= jnp.float32(5.0), jnp.float32(0.0),
                                 sw))
        sw = jnp.maximum(sw, jnp.float32(1e-30))
        zif = zi.astype(jnp.float32)
        zjf = zj.astype(jnp.float32)
        rep = (jnp.float32(0.5) * zif * zjf) * phi * sw * y
        repb[pl.ds(b16, L)] = rep

    def do_chunk(base, csize):
        pltpu.sync_copy(ii_hbm.at[pl.ds(base, csize)],
                        iib.at[pl.ds(0, csize)])
        pltpu.sync_copy(ij_hbm.at[pl.ds(base, csize)],
                        ijb.at[pl.ds(0, csize)])
        pltpu.sync_copy(d2_hbm.at[pl.ds(base, csize)],
                        db.at[pl.ds(0, csize)])

        def vec_body(j, carry):
            compute_vec(j)
            return carry

        lax.fori_loop(0, csize // L, vec_body, 0)
        if csize < CHUNK:
            zf = jnp.zeros((L,), jnp.float32)
            zidx = jnp.zeros((L,), jnp.int32)
            for t in range((CHUNK - csize) // L):
                off = csize + t * L
                repb[pl.ds(off, L)] = zf
                iib[pl.ds(off, L)] = zidx
        pltpu.sync_copy(repb, acc.at[iib], add=True)

    def chunk_body(c, carry):
        do_chunk(base0 + c * CHUNK, CHUNK)
        return carry

    lax.fori_loop(0, nfull, chunk_body, 0)
    if tail:
        do_chunk(base0 + nfull * CHUNK, tail)

    plsc.subcore_barrier()

    @pl.when(sid == 0)
    def _():
        pltpu.sync_copy(acc, out_hbm.at[cid])


def kernel(atomic_numbers, displacements, idx_i, idx_j, atom_mask,
           batch_segments, batch_mask, batch_size):
    n_nodes = atomic_numbers.shape[0]
    n_edges = idx_i.shape[0]
    zn = atomic_numbers.astype(jnp.int32)
    d2 = _edge_d2(displacements)
    zat = jnp.power(jnp.arange(128, dtype=jnp.float32), jnp.float32(0.23))
    zeros_nodes = jnp.zeros((n_nodes,), jnp.float32)

    body = functools.partial(_zbl_body, n_nodes=n_nodes, n_edges=n_edges)
    run = pl.kernel(
        body,
        mesh=plsc.VectorSubcoreMesh(core_axis_name="c", subcore_axis_name="s"),
        out_type=jax.ShapeDtypeStruct((NC, n_nodes), jnp.float32),
        compiler_params=pltpu.CompilerParams(needs_layout_passes=False),
        scratch_types=[
            pltpu.VMEM((n_nodes,), jnp.int32),
            pltpu.VMEM((128,), jnp.float32),
            pltpu.VMEM((CHUNK,), jnp.int32),
            pltpu.VMEM((CHUNK,), jnp.int32),
            pltpu.VMEM((CHUNK,), jnp.float32),
            pltpu.VMEM((CHUNK,), jnp.float32),
            pltpu.VMEM_SHARED((n_nodes,), jnp.float32),
        ],
    )
    partial = run(zn, d2, idx_i.astype(jnp.int32),
                  idx_j.astype(jnp.int32), zat, zeros_nodes)
    erep = (partial[0] + partial[1]) * atom_mask
    return erep[..., None, None, None]


# speedup vs baseline: 73.9061x; 1.4930x over previous
import functools

import jax
import jax.numpy as jnp
import numpy as np
from jax import lax
from jax.experimental import pallas as pl
from jax.experimental.pallas import tpu as pltpu
from jax.experimental.pallas import tpu_sc as plsc

NC = 2
NS = 16
L = 16
CHUNK = 2048

_PHI_C = np.abs(np.array([0.18175, 0.50986, 0.28022, 0.02817], np.float32))
_PHI_E = np.abs(np.array([3.1998, 0.94229, 0.4029, 0.20162], np.float32))
_SOFT = np.exp(_PHI_C - np.max(_PHI_C))
_COEF = (_SOFT / np.sum(_SOFT)).astype(np.float32)
_AEXP = (_PHI_E - _PHI_E[3]).astype(np.float32)
_INV_A = np.float32(1.0) / np.float32(0.8854)


def _d2_tc_kernel(x_ref, o_ref):
    x = x_ref[...]
    xt = jnp.swapaxes(x * x, 0, 1)
    o_ref[...] = xt[0] + xt[1] + xt[2]


def _edge_d2(displacements):
    n_edges = displacements.shape[0]
    blk = 16384
    return pl.pallas_call(
        _d2_tc_kernel,
        grid=(n_edges // blk,),
        in_specs=[pl.BlockSpec((blk, 3), lambda i: (i, 0))],
        out_specs=pl.BlockSpec((blk,), lambda i: (i,)),
        out_shape=jax.ShapeDtypeStruct((n_edges,), jnp.float32),
    )(displacements)


def _zbl_body(zn_hbm, d2_hbm, ii_hbm, ij_hbm, zat_hbm, zero_hbm, out_hbm,
              ztab, zatab, iib, ijb, db, repb, acc, n_nodes, n_edges):
    cid = lax.axis_index("c")
    sid = lax.axis_index("s")
    wid = cid * NS + sid
    ept = n_edges // (NC * NS)
    nfull = ept // CHUNK
    tail = ept - nfull * CHUNK
    base0 = wid * ept

    pltpu.sync_copy(zn_hbm, ztab)
    pltpu.sync_copy(zat_hbm, zatab)

    @pl.when(sid == 0)
    def _():
        pltpu.sync_copy(zero_hbm, acc)

    plsc.subcore_barrier()

    def compute_vec(j):
        b16 = j * L
        ii = iib[pl.ds(b16, L)]
        ij = ijb[pl.ds(b16, L)]
        zi = plsc.load_gather(ztab, [ii])
        zj = plsc.load_gather(ztab, [ij])
        zai = plsc.load_gather(zatab, [zi])
        zaj = plsc.load_gather(zatab, [zj])
        d2 = jnp.maximum(db[pl.ds(b16, L)], jnp.float32(1e-20))
        bits = lax.bitcast_convert_type(d2, jnp.int32)
        y = lax.bitcast_convert_type(
            jnp.int32(0x5F3759DF) - lax.shift_right_arithmetic(bits, 1),
            jnp.float32)
        half = jnp.float32(0.5) * d2
        for _ in range(3):
            y = y * (jnp.float32(1.5) - half * y * y)
        dist = d2 * y
        arg = dist * (zai + zaj) * _INV_A
        phi = (_COEF[0] * jnp.exp(-_AEXP[0] * arg)
               + _COEF[1] * jnp.exp(-_AEXP[1] * arg)
               + _COEF[2] * jnp.exp(-_AEXP[2] * arg)
               + _COEF[3])
        x = jnp.float32(5.0) - dist
        sw = ((jnp.float32(6.0) * x - jnp.float32(15.0)) * x
              + jnp.float32(10.0)) * x * x * x
        sw = jnp.where(dist < jnp.float32(4.0), jnp.float32(1.0),
                       jnp.where(dist >= jnp.float32(5.0), jnp.float32(0.0),
                                 sw))
        sw = jnp.maximum(sw, jnp.float32(1e-30))
        zif = zi.astype(jnp.float32)
        zjf = zj.astype(jnp.float32)
        rep = (jnp.float32(0.5) * zif * zjf) * phi * sw * y
        repb[pl.ds(b16, L)] = rep

    def do_chunk(base, csize):
        pltpu.sync_copy(ii_hbm.at[pl.ds(base, csize)],
                        iib.at[pl.ds(0, csize)])
        pltpu.sync_copy(ij_hbm.at[pl.ds(base, csize)],
                        ijb.at[pl.ds(0, csize)])
        pltpu.sync_copy(d2_hbm.at[pl.ds(base, csize)],
                        db.at[pl.ds(0, csize)])

        def vec_body(j, carry):
            compute_vec(j)
            return carry

        lax.fori_loop(0, csize // L, vec_body, 0)
        if csize < CHUNK:
            zf = jnp.zeros((L,), jnp.float32)
            zidx = jnp.zeros((L,), jnp.int32)
            for t in range((CHUNK - csize) // L):
                off = csize + t * L
                repb[pl.ds(off, L)] = zf
                iib[pl.ds(off, L)] = zidx
        pltpu.sync_copy(repb, acc.at[iib], add=True)

    def chunk_body(c, carry):
        do_chunk(base0 + c * CHUNK, CHUNK)
        return carry

    lax.fori_loop(0, nfull, chunk_body, 0)
    if tail:
        do_chunk(base0 + nfull * CHUNK, tail)

    plsc.subcore_barrier()

    @pl.when(sid == 0)
    def _():
        pltpu.sync_copy(acc, out_hbm.at[cid])


def kernel(atomic_numbers, displacements, idx_i, idx_j, atom_mask,
           batch_segments, batch_mask, batch_size):
    n_nodes = atomic_numbers.shape[0]
    n_edges = idx_i.shape[0]
    zn = atomic_numbers.astype(jnp.int32)
    d2 = _edge_d2(displacements)
    zat = jnp.power(jnp.arange(128, dtype=jnp.float32), jnp.float32(0.23))
    zeros_nodes = jnp.zeros((n_nodes,), jnp.float32)

    body = functools.partial(_zbl_body, n_nodes=n_nodes, n_edges=n_edges)
    run = pl.kernel(
        body,
        mesh=plsc.VectorSubcoreMesh(core_axis_name="c", subcore_axis_name="s"),
        out_type=jax.ShapeDtypeStruct((NC, n_nodes), jnp.float32),
        compiler_params=pltpu.CompilerParams(needs_layout_passes=False),
        scratch_types=[
            pltpu.VMEM((n_nodes,), jnp.int32),
            pltpu.VMEM((128,), jnp.float32),
            pltpu.VMEM((CHUNK,), jnp.int32),
            pltpu.VMEM((CHUNK,), jnp.int32),
            pltpu.VMEM((CHUNK,), jnp.float32),
            pltpu.VMEM((CHUNK,), jnp.float32),
            pltpu.VMEM_SHARED((n_nodes,), jnp.float32),
        ],
    )
    partial = run(zn, d2, idx_i.astype(jnp.int32),
                  idx_j.astype(jnp.int32), zat, zeros_nodes)
    erep = (partial[0] + partial[1]) * atom_mask
    return erep[..., None, None, None]


# speedup vs baseline: 244.9976x; 3.3150x over previous
import functools

import jax
import jax.numpy as jnp
import numpy as np
from jax import lax
from jax.experimental import pallas as pl
from jax.experimental.pallas import tpu as pltpu
from jax.experimental.pallas import tpu_sc as plsc

NC = 2
NS = 16
L = 16
CHUNK = 2048

_PHI_C = np.abs(np.array([0.18175, 0.50986, 0.28022, 0.02817], np.float32))
_PHI_E = np.abs(np.array([3.1998, 0.94229, 0.4029, 0.20162], np.float32))
_SOFT = np.exp(_PHI_C - np.max(_PHI_C))
_COEF = (_SOFT / np.sum(_SOFT)).astype(np.float32)
_AEXP = (_PHI_E - _PHI_E[3]).astype(np.float32)
_INV_A = np.float32(1.0) / np.float32(0.8854)


def _zbl_body(zn_hbm, dx_hbm, dy_hbm, dz_hbm, ii_hbm, ij_hbm, zat_hbm,
              zero_hbm, out_hbm, ztab, zatab, iib, ijb, dxb, dyb, dzb, repb,
              acc, n_nodes, n_edges):
    cid = lax.axis_index("c")
    sid = lax.axis_index("s")
    wid = cid * NS + sid
    ept = n_edges // (NC * NS)
    nfull = ept // CHUNK
    tail = ept - nfull * CHUNK
    base0 = wid * ept

    pltpu.sync_copy(zn_hbm, ztab)
    pltpu.sync_copy(zat_hbm, zatab)

    @pl.when(sid == 0)
    def _():
        pltpu.sync_copy(zero_hbm, acc)

    plsc.subcore_barrier()

    def compute_vec(j):
        b16 = j * L
        ii = iib[pl.ds(b16, L)]
        ij = ijb[pl.ds(b16, L)]
        zi = plsc.load_gather(ztab, [ii])
        zj = plsc.load_gather(ztab, [ij])
        zai = plsc.load_gather(zatab, [zi])
        zaj = plsc.load_gather(zatab, [zj])
        dx = dxb[pl.ds(b16, L)]
        dy = dyb[pl.ds(b16, L)]
        dz = dzb[pl.ds(b16, L)]
        d2 = jnp.maximum(dx * dx + dy * dy + dz * dz, jnp.float32(1e-20))
        bits = lax.bitcast_convert_type(d2, jnp.int32)
        y = lax.bitcast_convert_type(
            jnp.int32(0x5F3759DF) - lax.shift_right_arithmetic(bits, 1),
            jnp.float32)
        half = jnp.float32(0.5) * d2
        for _ in range(3):
            y = y * (jnp.float32(1.5) - half * y * y)
        dist = d2 * y
        arg = dist * (zai + zaj) * _INV_A
        phi = (_COEF[0] * jnp.exp(-_AEXP[0] * arg)
               + _COEF[1] * jnp.exp(-_AEXP[1] * arg)
               + _COEF[2] * jnp.exp(-_AEXP[2] * arg)
               + _COEF[3])
        x = jnp.float32(5.0) - dist
        sw = ((jnp.float32(6.0) * x - jnp.float32(15.0)) * x
              + jnp.float32(10.0)) * x * x * x
        sw = jnp.where(dist < jnp.float32(4.0), jnp.float32(1.0),
                       jnp.where(dist >= jnp.float32(5.0), jnp.float32(0.0),
                                 sw))
        sw = jnp.maximum(sw, jnp.float32(1e-30))
        zif = zi.astype(jnp.float32)
        zjf = zj.astype(jnp.float32)
        rep = (jnp.float32(0.5) * zif * zjf) * phi * sw * y
        repb[pl.ds(b16, L)] = rep

    def do_chunk(base, csize):
        pltpu.sync_copy(ii_hbm.at[pl.ds(base, csize)],
                        iib.at[pl.ds(0, csize)])
        pltpu.sync_copy(ij_hbm.at[pl.ds(base, csize)],
                        ijb.at[pl.ds(0, csize)])
        pltpu.sync_copy(dx_hbm.at[pl.ds(base, csize)],
                        dxb.at[pl.ds(0, csize)])
        pltpu.sync_copy(dy_hbm.at[pl.ds(base, csize)],
                        dyb.at[pl.ds(0, csize)])
        pltpu.sync_copy(dz_hbm.at[pl.ds(base, csize)],
                        dzb.at[pl.ds(0, csize)])

        def vec_body(j, carry):
            compute_vec(j)
            return carry

        lax.fori_loop(0, csize // L, vec_body, 0)
        if csize < CHUNK:
            zf = jnp.zeros((L,), jnp.float32)
            zidx = jnp.zeros((L,), jnp.int32)
            for t in range((CHUNK - csize) // L):
                off = csize + t * L
                repb[pl.ds(off, L)] = zf
                iib[pl.ds(off, L)] = zidx
        pltpu.sync_copy(repb, acc.at[iib], add=True)

    def chunk_body(c, carry):
        do_chunk(base0 + c * CHUNK, CHUNK)
        return carry

    lax.fori_loop(0, nfull, chunk_body, 0)
    if tail:
        do_chunk(base0 + nfull * CHUNK, tail)

    plsc.subcore_barrier()

    @pl.when(sid == 0)
    def _():
        pltpu.sync_copy(acc, out_hbm.at[cid])


def kernel(atomic_numbers, displacements, idx_i, idx_j, atom_mask,
           batch_segments, batch_mask, batch_size):
    n_nodes = atomic_numbers.shape[0]
    n_edges = idx_i.shape[0]
    zn = atomic_numbers.astype(jnp.int32)
    dx = displacements[:, 0]
    dy = displacements[:, 1]
    dz = displacements[:, 2]
    zat = jnp.power(jnp.arange(128, dtype=jnp.float32), jnp.float32(0.23))
    zeros_nodes = jnp.zeros((n_nodes,), jnp.float32)

    body = functools.partial(_zbl_body, n_nodes=n_nodes, n_edges=n_edges)
    run = pl.kernel(
        body,
        mesh=plsc.VectorSubcoreMesh(core_axis_name="c", subcore_axis_name="s"),
        out_type=jax.ShapeDtypeStruct((NC, n_nodes), jnp.float32),
        compiler_params=pltpu.CompilerParams(needs_layout_passes=False),
        scratch_types=[
            pltpu.VMEM((n_nodes,), jnp.int32),
            pltpu.VMEM((128,), jnp.float32),
            pltpu.VMEM((CHUNK,), jnp.int32),
            pltpu.VMEM((CHUNK,), jnp.int32),
            pltpu.VMEM((CHUNK,), jnp.float32),
            pltpu.VMEM((CHUNK,), jnp.float32),
            pltpu.VMEM((CHUNK,), jnp.float32),
            pltpu.VMEM((CHUNK,), jnp.float32),
            pltpu.VMEM_SHARED((n_nodes,), jnp.float32),
        ],
    )
    partial = run(zn, dx, dy, dz, idx_i.astype(jnp.int32),
                  idx_j.astype(jnp.int32), zat, zeros_nodes)
    erep = (partial[0] + partial[1]) * atom_mask
    return erep[..., None, None, None]
